# baseline (speedup 1.0000x reference)
import functools

import jax
import jax.numpy as jnp
from jax import lax
from jax.experimental import pallas as pl
from jax.experimental.pallas import tpu as pltpu
from jax.experimental.pallas import tpu_sc as plsc

_NC = 2
_NS = 16
_L = 16
_CK = 128


def _round_up(v, m):
    return (v + m - 1) // m * m


def kernel(x, edge_index, Wg0, bg0, Wfc0, bfc0, g0, be0, Wg1, bg1, Wfc1, bfc1, g1, be1):
    N, H = x.shape
    E = edge_index.shape[1]
    OUT = Wfc1.shape[1]
    HQ = H // 4
    R = _round_up(-(-N // _NS), 128)
    NP = R * _NS
    G = _NS
    JS = 7
    NB = 4
    JD = 4
    EP = _round_up(E, _NS * _CK * JS * NB)
    ROWS = EP // _CK
    ert = ROWS // _NS
    erd = ROWS // (_NC * _NS)
    NCH = ert // JS

    f32 = jnp.float32
    src = edge_index[0].astype(jnp.int32)
    dst = edge_index[1].astype(jnp.int32)
    pad = EP - E
    srcp = jnp.concatenate([src, jnp.zeros((pad,), jnp.int32)]).reshape(ROWS, _CK)
    dstp = jnp.concatenate([dst, jnp.full((pad,), N, jnp.int32)]).reshape(ROWS, _CK)
    zeros_np = jnp.zeros((NP,), f32)

    mesh = plsc.VectorSubcoreMesh(
        core_axis_name="c", subcore_axis_name="s", num_cores=_NC, num_subcores=_NS)
    sc_params = pltpu.CompilerParams(use_tc_tiling_on_sc=False)

    @functools.partial(
        pl.kernel,
        mesh=mesh,
        compiler_params=sc_params,
        out_type=jax.ShapeDtypeStruct((_NC * NP,), f32),
        scratch_types=[
            pltpu.VMEM((JD, _CK), jnp.int32),
            pltpu.VMEM((_CK,), f32),
            pltpu.VMEM_SHARED((NP,), f32),
        ],
    )
    def deg_kernel(dst_hbm, zeros_hbm, out_hbm, didx_v, ones_v, deg_sh):
        c = lax.axis_index("c")
        s = lax.axis_index("s")
        w = c * _NS + s
        for i in range(_CK // _L):
            ones_v[pl.ds(i * _L, _L)] = jnp.ones((_L,), f32)
        pltpu.sync_copy(zeros_hbm.at[pl.ds(s * R, R)], deg_sh.at[pl.ds(s * R, R)])
        plsc.subcore_barrier()

        def body(j, carry):
            r0 = w * erd + j * JD
            pltpu.sync_copy(dst_hbm.at[pl.ds(r0, JD)], didx_v)
            for k in range(JD):
                pltpu.sync_copy(ones_v, deg_sh.at[didx_v.at[k]], add=True)
            return carry

        lax.fori_loop(0, erd // JD, body, 0)
        plsc.subcore_barrier()
        pltpu.sync_copy(deg_sh.at[pl.ds(s * R, R)],
                        out_hbm.at[pl.ds(c * NP + s * R, R)])

    deg2 = deg_kernel(dstp, zeros_np).reshape(_NC, NP)

    @functools.partial(
        pl.kernel,
        mesh=mesh,
        compiler_params=sc_params,
        out_type=[jax.ShapeDtypeStruct((NP, HQ), f32) for _ in range(4)],
        scratch_types=[
            pltpu.VMEM((NB, JS, _CK), jnp.int32),
            pltpu.VMEM((NB, JS, _CK), jnp.int32),
            pltpu.VMEM((NB, JS, _CK, HQ), f32),
            pltpu.VMEM_SHARED((NP, HQ), f32),
        ] + [pltpu.SemaphoreType.DMA] * (2 * NB),
    )
    def scat_kernel(hs0_hbm, hs1_hbm, hs2_hbm, hs3_hbm, src_hbm, dst_hbm,
                    out0_hbm, out1_hbm, out2_hbm, out3_hbm,
                    sidx_v, didx_v, rows_v, acc_sh, *sems):
        c = lax.axis_index("c")
        s = lax.axis_index("s")
        hs_all = (hs0_hbm, hs1_hbm, hs2_hbm, hs3_hbm)
        out_all = (out0_hbm, out1_hbm, out2_hbm, out3_hbm)
        gsems = sems[:NB]
        ssems = sems[NB:]

        for kk in range(_NC):

            @pl.when(c == kk)
            def _(kk=kk):
                for q in (2 * kk, 2 * kk + 1):
                    hs_hbm = hs_all[q]
                    out_hbm = out_all[q]

                    def load_idx(j, b):
                        r0 = s * ert + j * JS
                        pltpu.sync_copy(src_hbm.at[pl.ds(r0, JS)], sidx_v.at[b])
                        pltpu.sync_copy(dst_hbm.at[pl.ds(r0, JS)], didx_v.at[b])

                    def fire_gathers(b, hs_hbm=hs_hbm):
                        for k in range(JS):
                            pltpu.async_copy(hs_hbm.at[sidx_v.at[b, k]],
                                             rows_v.at[b, k], gsems[b])

                    def wait_gathers(b, hs_hbm=hs_hbm):
                        for k in range(JS):
                            pltpu.make_async_copy(hs_hbm.at[sidx_v.at[b, k]],
                                                  rows_v.at[b, k], gsems[b]).wait()

                    def fire_scatters(b):
                        for k in range(JS):
                            pltpu.async_copy(rows_v.at[b, k],
                                             acc_sh.at[didx_v.at[b, k]],
                                             ssems[b], add=True)

                    def wait_scatters(b):
                        for k in range(JS):
                            pltpu.make_async_copy(rows_v.at[b, k],
                                                  acc_sh.at[didx_v.at[b, k]],
                                                  ssems[b]).wait()

                    pltpu.sync_copy(hs_hbm.at[pl.ds(s * R, R)],
                                    acc_sh.at[pl.ds(s * R, R)])
                    plsc.subcore_barrier()

                    for b0 in range(NB - 1):
                        load_idx(b0, b0)
                        fire_gathers(b0)

                    def body(i, carry):
                        for b in range(NB):
                            j = NB * i + b
                            nb = (b + NB - 1) % NB

                            @pl.when(j + NB - 1 < NCH)
                            def _(j=j, nb=nb):
                                @pl.when(j >= 1)
                                def _():
                                    wait_scatters(nb)
                                load_idx(j + NB - 1, nb)
                                fire_gathers(nb)

                            wait_gathers(b)
                            fire_scatters(b)
                        return carry

                    lax.fori_loop(0, NCH // NB, body, 0)
                    for b0 in range(NB):
                        wait_scatters(b0)
                    plsc.subcore_barrier()
                    pltpu.sync_copy(acc_sh.at[pl.ds(s * R, R)],
                                    out_hbm.at[pl.ds(s * R, R)])

    def _head(deg_ref, x_ref, w_ref, hs0_ref, hs1_ref, hs2_ref, hs3_ref, dinv_ref):
        deg = jnp.sum(deg_ref[...], axis=0) + 1.0
        dinv = lax.rsqrt(deg)
        h = jnp.dot(x_ref[...], w_ref[...], preferred_element_type=f32)
        hs = h * dinv[:, None]
        for q, ref in enumerate((hs0_ref, hs1_ref, hs2_ref, hs3_ref)):
            ref[...] = hs[:, q * HQ:(q + 1) * HQ]
        dinv_ref[...] = dinv[:, None]

    _hs_out_specs = [pl.BlockSpec((R, HQ), lambda i: (i, 0)) for _ in range(4)]
    _hs_out_shape = [jax.ShapeDtypeStruct((NP, HQ), f32) for _ in range(4)]

    head = pl.pallas_call(
        _head,
        grid=(G,),
        in_specs=[
            pl.BlockSpec((_NC, R), lambda i: (0, i)),
            pl.BlockSpec((R, H), lambda i: (i, 0)),
            pl.BlockSpec((H, H), lambda i: (0, 0)),
        ],
        out_specs=_hs_out_specs + [pl.BlockSpec((R, 1), lambda i: (i, 0))],
        out_shape=_hs_out_shape + [jax.ShapeDtypeStruct((NP, 1), f32)],
    )

    _acc_spec = pl.BlockSpec((R, HQ), lambda i: (jnp.where(i < G, i, 0), 0))
    _row_spec = pl.BlockSpec((R, 1), lambda i: (i % G, 0))

    def _phaseA(i, acc_refs, dinv_ref, bg_ref, wfc_ref, bfc_ref, z_s, ps_s, psq_s):
        @pl.when(i == 0)
        def _():
            ps_s[...] = jnp.zeros_like(ps_s[...])
            psq_s[...] = jnp.zeros_like(psq_s[...])

        @pl.when(i < G)
        def _():
            acc = jnp.concatenate([r[...] for r in acc_refs], axis=1)
            conv = acc * dinv_ref[...] + bg_ref[...]
            z = jnp.dot(conv, wfc_ref[...], preferred_element_type=f32) + bfc_ref[...]
            rid = i * R + lax.broadcasted_iota(jnp.int32, (R, 1), 0)
            zm = jnp.where(rid < N, z, 0.0)
            z_s[i] = z
            ps_s[...] += jnp.sum(zm, axis=0, keepdims=True)
            psq_s[...] += jnp.sum(zm * zm, axis=0, keepdims=True)

    def _bn_x(i, g_ref, be_ref, z_s, ps_s, psq_s):
        mu = ps_s[...] * (1.0 / N)
        ex2 = psq_s[...] * (1.0 / N)
        istd = lax.rsqrt(ex2 - mu * mu + 1e-5)
        return jnp.maximum((z_s[i - G] - mu) * istd * g_ref[...] + be_ref[...], 0.0)

    def _mid(acc0_ref, acc1_ref, acc2_ref, acc3_ref, dinv_ref, bg_ref, wfc_ref,
             bfc_ref, g_ref, be_ref, wg_ref, hs0_ref, hs1_ref, hs2_ref, hs3_ref,
             z_s, ps_s, psq_s):
        i = pl.program_id(0)
        _phaseA(i, (acc0_ref, acc1_ref, acc2_ref, acc3_ref), dinv_ref, bg_ref,
                wfc_ref, bfc_ref, z_s, ps_s, psq_s)

        @pl.when(i >= G)
        def _():
            x1 = _bn_x(i, g_ref, be_ref, z_s, ps_s, psq_s)
            h = jnp.dot(x1, wg_ref[...], preferred_element_type=f32)
            hs = h * dinv_ref[...]
            for q, ref in enumerate((hs0_ref, hs1_ref, hs2_ref, hs3_ref)):
                ref[...] = hs[:, q * HQ:(q + 1) * HQ]

    _hs_out_specs2 = [
        pl.BlockSpec((R, HQ), lambda i: (jnp.where(i < G, 0, i - G), 0))
        for _ in range(4)
    ]
    mid = pl.pallas_call(
        _mid,
        grid=(2 * G,),
        in_specs=[
            _acc_spec, _acc_spec, _acc_spec, _acc_spec,
            _row_spec,
            pl.BlockSpec((1, H), lambda i: (0, 0)),
            pl.BlockSpec((H, H), lambda i: (0, 0)),
            pl.BlockSpec((1, H), lambda i: (0, 0)),
            pl.BlockSpec((1, H), lambda i: (0, 0)),
            pl.BlockSpec((1, H), lambda i: (0, 0)),
            pl.BlockSpec((H, H), lambda i: (0, 0)),
        ],
        out_specs=_hs_out_specs2,
        out_shape=_hs_out_shape,
        scratch_shapes=[
            pltpu.VMEM((G, R, H), f32),
            pltpu.VMEM((1, H), f32),
            pltpu.VMEM((1, H), f32),
        ],
    )

    def _tail(acc0_ref, acc1_ref, acc2_ref, acc3_ref, dinv_ref, bg_ref, wfc_ref,
              bfc_ref, g_ref, be_ref, o_ref, z_s, ps_s, psq_s):
        i = pl.program_id(0)
        _phaseA(i, (acc0_ref, acc1_ref, acc2_ref, acc3_ref), dinv_ref, bg_ref,
                wfc_ref, bfc_ref, z_s, ps_s, psq_s)

        @pl.when(i >= G)
        def _():
            o_ref[...] = _bn_x(i, g_ref, be_ref, z_s, ps_s, psq_s)

    tail = pl.pallas_call(
        _tail,
        grid=(2 * G,),
        in_specs=[
            _acc_spec, _acc_spec, _acc_spec, _acc_spec,
            _row_spec,
            pl.BlockSpec((1, H), lambda i: (0, 0)),
            pl.BlockSpec((H, OUT), lambda i: (0, 0)),
            pl.BlockSpec((1, OUT), lambda i: (0, 0)),
            pl.BlockSpec((1, OUT), lambda i: (0, 0)),
            pl.BlockSpec((1, OUT), lambda i: (0, 0)),
        ],
        out_specs=pl.BlockSpec((R, OUT), lambda i: (jnp.where(i < G, 0, i - G), 0)),
        out_shape=jax.ShapeDtypeStruct((NP, OUT), f32),
        scratch_shapes=[
            pltpu.VMEM((G, R, OUT), f32),
            pltpu.VMEM((1, OUT), f32),
            pltpu.VMEM((1, OUT), f32),
        ],
    )

    bg0r, bfc0r, g0r, be0r = (v.reshape(1, -1) for v in (bg0, bfc0, g0, be0))
    bg1r, bfc1r, g1r, be1r = (v.reshape(1, -1) for v in (bg1, bfc1, g1, be1))

    hs0 = head(deg2, x, Wg0)
    hs0, dinv = hs0[:4], hs0[4]
    acc0 = scat_kernel(*hs0, srcp, dstp)
    hs1 = mid(*acc0, dinv, bg0r, Wfc0, bfc0r, g0r, be0r, Wg1)
    acc1 = scat_kernel(*hs1, srcp, dstp)
    out = tail(*acc1, dinv, bg1r, Wfc1, bfc1r, g1r, be1r)
    return out[:N]


# speedup vs baseline: 1.1173x; 1.1173x over previous
import functools

import jax
import jax.numpy as jnp
from jax import lax
from jax.experimental import pallas as pl
from jax.experimental.pallas import tpu as pltpu
from jax.experimental.pallas import tpu_sc as plsc

_NC = 2
_NS = 16
_L = 16
_CK = 128


def _round_up(v, m):
    return (v + m - 1) // m * m


def kernel(x, edge_index, Wg0, bg0, Wfc0, bfc0, g0, be0, Wg1, bg1, Wfc1, bfc1, g1, be1):
    N, H = x.shape
    E = edge_index.shape[1]
    OUT = Wfc1.shape[1]
    HQ = H // 4
    R = _round_up(-(-N // _NS), 128)
    NP = R * _NS
    G = _NS
    JS = 14
    NB = 2
    JD = 4
    EP = _round_up(E, _NS * _CK * JS * NB)
    ROWS = EP // _CK
    ert = ROWS // _NS
    erd = ROWS // (_NC * _NS)
    NCH = ert // JS

    f32 = jnp.float32
    ei2d = jnp.pad(edge_index.astype(jnp.int32), ((0, 0), (0, EP - E)),
                   constant_values=N).reshape(2 * ROWS, _CK)
    zeros_np = jnp.zeros((NP,), f32)

    mesh = plsc.VectorSubcoreMesh(
        core_axis_name="c", subcore_axis_name="s", num_cores=_NC, num_subcores=_NS)
    sc_params = pltpu.CompilerParams(use_tc_tiling_on_sc=False)

    @functools.partial(
        pl.kernel,
        mesh=mesh,
        compiler_params=sc_params,
        out_type=jax.ShapeDtypeStruct((_NC * NP,), f32),
        scratch_types=[
            pltpu.VMEM((JD, _CK), jnp.int32),
            pltpu.VMEM((_CK,), f32),
            pltpu.VMEM_SHARED((NP,), f32),
        ],
    )
    def deg_kernel(ei_hbm, zeros_hbm, out_hbm, didx_v, ones_v, deg_sh):
        c = lax.axis_index("c")
        s = lax.axis_index("s")
        w = c * _NS + s
        for i in range(_CK // _L):
            ones_v[pl.ds(i * _L, _L)] = jnp.ones((_L,), f32)
        pltpu.sync_copy(zeros_hbm.at[pl.ds(s * R, R)], deg_sh.at[pl.ds(s * R, R)])
        plsc.subcore_barrier()

        def body(j, carry):
            r0 = ROWS + w * erd + j * JD
            pltpu.sync_copy(ei_hbm.at[pl.ds(r0, JD)], didx_v)
            for k in range(JD):
                pltpu.sync_copy(ones_v, deg_sh.at[didx_v.at[k]], add=True)
            return carry

        lax.fori_loop(0, erd // JD, body, 0)
        plsc.subcore_barrier()
        pltpu.sync_copy(deg_sh.at[pl.ds(s * R, R)],
                        out_hbm.at[pl.ds(c * NP + s * R, R)])

    deg2 = deg_kernel(ei2d, zeros_np).reshape(_NC, NP)

    @functools.partial(
        pl.kernel,
        mesh=mesh,
        compiler_params=sc_params,
        out_type=[jax.ShapeDtypeStruct((NP, HQ), f32) for _ in range(4)],
        scratch_types=[
            pltpu.VMEM((NB, JS, _CK), jnp.int32),
            pltpu.VMEM((NB, JS, _CK), jnp.int32),
            pltpu.VMEM((NB, JS, _CK, HQ), f32),
            pltpu.VMEM_SHARED((NP, HQ), f32),
        ] + [pltpu.SemaphoreType.DMA] * (2 * NB),
    )
    def scat_kernel(hs0_hbm, hs1_hbm, hs2_hbm, hs3_hbm, ei_hbm,
                    out0_hbm, out1_hbm, out2_hbm, out3_hbm,
                    sidx_v, didx_v, rows_v, acc_sh, *sems):
        c = lax.axis_index("c")
        s = lax.axis_index("s")
        hs_all = (hs0_hbm, hs1_hbm, hs2_hbm, hs3_hbm)
        out_all = (out0_hbm, out1_hbm, out2_hbm, out3_hbm)
        gsems = sems[:NB]
        ssems = sems[NB:]

        for kk in range(_NC):

            @pl.when(c == kk)
            def _(kk=kk):
                for q in (2 * kk, 2 * kk + 1):
                    hs_hbm = hs_all[q]
                    out_hbm = out_all[q]

                    def load_idx(j, b):
                        r0 = s * ert + j * JS
                        pltpu.sync_copy(ei_hbm.at[pl.ds(r0, JS)], sidx_v.at[b])
                        pltpu.sync_copy(ei_hbm.at[pl.ds(ROWS + r0, JS)], didx_v.at[b])

                    def fire_gathers(b, hs_hbm=hs_hbm):
                        for k in range(JS):
                            pltpu.async_copy(hs_hbm.at[sidx_v.at[b, k]],
                                             rows_v.at[b, k], gsems[b])

                    def wait_gathers(b, hs_hbm=hs_hbm):
                        for k in range(JS):
                            pltpu.make_async_copy(hs_hbm.at[sidx_v.at[b, k]],
                                                  rows_v.at[b, k], gsems[b]).wait()

                    def fire_scatters(b):
                        for k in range(JS):
                            pltpu.async_copy(rows_v.at[b, k],
                                             acc_sh.at[didx_v.at[b, k]],
                                             ssems[b], add=True)

                    def wait_scatters(b):
                        for k in range(JS):
                            pltpu.make_async_copy(rows_v.at[b, k],
                                                  acc_sh.at[didx_v.at[b, k]],
                                                  ssems[b]).wait()

                    pltpu.sync_copy(hs_hbm.at[pl.ds(s * R, R)],
                                    acc_sh.at[pl.ds(s * R, R)])
                    plsc.subcore_barrier()

                    for b0 in range(NB - 1):
                        load_idx(b0, b0)
                        fire_gathers(b0)

                    def body(i, carry):
                        for b in range(NB):
                            j = NB * i + b
                            nb = (b + NB - 1) % NB

                            @pl.when(j + NB - 1 < NCH)
                            def _(j=j, nb=nb):
                                @pl.when(j >= 1)
                                def _():
                                    wait_scatters(nb)
                                load_idx(j + NB - 1, nb)
                                fire_gathers(nb)

                            wait_gathers(b)
                            fire_scatters(b)
                        return carry

                    lax.fori_loop(0, NCH // NB, body, 0)
                    for b0 in range(NB):
                        wait_scatters(b0)
                    plsc.subcore_barrier()
                    pltpu.sync_copy(acc_sh.at[pl.ds(s * R, R)],
                                    out_hbm.at[pl.ds(s * R, R)])

    def _head(deg_ref, x_ref, w_ref, hs0_ref, hs1_ref, hs2_ref, hs3_ref, dinv_ref):
        deg = jnp.sum(deg_ref[...], axis=0) + 1.0
        dinv = lax.rsqrt(deg)
        h = jnp.dot(x_ref[...], w_ref[...], preferred_element_type=f32)
        hs = h * dinv[:, None]
        for q, ref in enumerate((hs0_ref, hs1_ref, hs2_ref, hs3_ref)):
            ref[...] = hs[:, q * HQ:(q + 1) * HQ]
        dinv_ref[...] = dinv[:, None]

    _hs_out_specs = [pl.BlockSpec((R, HQ), lambda i: (i, 0)) for _ in range(4)]
    _hs_out_shape = [jax.ShapeDtypeStruct((NP, HQ), f32) for _ in range(4)]

    head = pl.pallas_call(
        _head,
        grid=(G,),
        in_specs=[
            pl.BlockSpec((_NC, R), lambda i: (0, i)),
            pl.BlockSpec((R, H), lambda i: (i, 0)),
            pl.BlockSpec((H, H), lambda i: (0, 0)),
        ],
        out_specs=_hs_out_specs + [pl.BlockSpec((R, 1), lambda i: (i, 0))],
        out_shape=_hs_out_shape + [jax.ShapeDtypeStruct((NP, 1), f32)],
    )

    _acc_spec = pl.BlockSpec((R, HQ), lambda i: (jnp.where(i < G, i, 0), 0))
    _row_spec = pl.BlockSpec((R, 1), lambda i: (i % G, 0))

    def _phaseA(i, acc_refs, dinv_ref, bg_ref, wfc_ref, bfc_ref, z_s, ps_s, psq_s):
        @pl.when(i == 0)
        def _():
            ps_s[...] = jnp.zeros_like(ps_s[...])
            psq_s[...] = jnp.zeros_like(psq_s[...])

        @pl.when(i < G)
        def _():
            acc = jnp.concatenate([r[...] for r in acc_refs], axis=1)
            conv = acc * dinv_ref[...] + bg_ref[...]
            z = jnp.dot(conv, wfc_ref[...], preferred_element_type=f32) + bfc_ref[...]
            rid = i * R + lax.broadcasted_iota(jnp.int32, (R, 1), 0)
            zm = jnp.where(rid < N, z, 0.0)
            z_s[i] = z
            ps_s[...] += jnp.sum(zm, axis=0, keepdims=True)
            psq_s[...] += jnp.sum(zm * zm, axis=0, keepdims=True)

    def _bn_x(i, g_ref, be_ref, z_s, ps_s, psq_s):
        mu = ps_s[...] * (1.0 / N)
        ex2 = psq_s[...] * (1.0 / N)
        istd = lax.rsqrt(ex2 - mu * mu + 1e-5)
        return jnp.maximum((z_s[i - G] - mu) * istd * g_ref[...] + be_ref[...], 0.0)

    def _mid(acc0_ref, acc1_ref, acc2_ref, acc3_ref, dinv_ref, bg_ref, wfc_ref,
             bfc_ref, g_ref, be_ref, wg_ref, hs0_ref, hs1_ref, hs2_ref, hs3_ref,
             z_s, ps_s, psq_s):
        i = pl.program_id(0)
        _phaseA(i, (acc0_ref, acc1_ref, acc2_ref, acc3_ref), dinv_ref, bg_ref,
                wfc_ref, bfc_ref, z_s, ps_s, psq_s)

        @pl.when(i >= G)
        def _():
            x1 = _bn_x(i, g_ref, be_ref, z_s, ps_s, psq_s)
            h = jnp.dot(x1, wg_ref[...], preferred_element_type=f32)
            hs = h * dinv_ref[...]
            for q, ref in enumerate((hs0_ref, hs1_ref, hs2_ref, hs3_ref)):
                ref[...] = hs[:, q * HQ:(q + 1) * HQ]

    _hs_out_specs2 = [
        pl.BlockSpec((R, HQ), lambda i: (jnp.where(i < G, 0, i - G), 0))
        for _ in range(4)
    ]
    mid = pl.pallas_call(
        _mid,
        grid=(2 * G,),
        in_specs=[
            _acc_spec, _acc_spec, _acc_spec, _acc_spec,
            _row_spec,
            pl.BlockSpec((1, H), lambda i: (0, 0)),
            pl.BlockSpec((H, H), lambda i: (0, 0)),
            pl.BlockSpec((1, H), lambda i: (0, 0)),
            pl.BlockSpec((1, H), lambda i: (0, 0)),
            pl.BlockSpec((1, H), lambda i: (0, 0)),
            pl.BlockSpec((H, H), lambda i: (0, 0)),
        ],
        out_specs=_hs_out_specs2,
        out_shape=_hs_out_shape,
        scratch_shapes=[
            pltpu.VMEM((G, R, H), f32),
            pltpu.VMEM((1, H), f32),
            pltpu.VMEM((1, H), f32),
        ],
    )

    def _tail(acc0_ref, acc1_ref, acc2_ref, acc3_ref, dinv_ref, bg_ref, wfc_ref,
              bfc_ref, g_ref, be_ref, o_ref, z_s, ps_s, psq_s):
        i = pl.program_id(0)
        _phaseA(i, (acc0_ref, acc1_ref, acc2_ref, acc3_ref), dinv_ref, bg_ref,
                wfc_ref, bfc_ref, z_s, ps_s, psq_s)

        @pl.when(i >= G)
        def _():
            o_ref[...] = _bn_x(i, g_ref, be_ref, z_s, ps_s, psq_s)

    tail = pl.pallas_call(
        _tail,
        grid=(2 * G,),
        in_specs=[
            _acc_spec, _acc_spec, _acc_spec, _acc_spec,
            _row_spec,
            pl.BlockSpec((1, H), lambda i: (0, 0)),
            pl.BlockSpec((H, OUT), lambda i: (0, 0)),
            pl.BlockSpec((1, OUT), lambda i: (0, 0)),
            pl.BlockSpec((1, OUT), lambda i: (0, 0)),
            pl.BlockSpec((1, OUT), lambda i: (0, 0)),
        ],
        out_specs=pl.BlockSpec((R, OUT), lambda i: (jnp.where(i < G, 0, i - G), 0)),
        out_shape=jax.ShapeDtypeStruct((N, OUT), f32),
        scratch_shapes=[
            pltpu.VMEM((G, R, OUT), f32),
            pltpu.VMEM((1, OUT), f32),
            pltpu.VMEM((1, OUT), f32),
        ],
    )

    bg0r, bfc0r, g0r, be0r = (v.reshape(1, -1) for v in (bg0, bfc0, g0, be0))
    bg1r, bfc1r, g1r, be1r = (v.reshape(1, -1) for v in (bg1, bfc1, g1, be1))

    hs0 = head(deg2, x, Wg0)
    hs0, dinv = hs0[:4], hs0[4]
    acc0 = scat_kernel(*hs0, ei2d)
    hs1 = mid(*acc0, dinv, bg0r, Wfc0, bfc0r, g0r, be0r, Wg1)
    acc1 = scat_kernel(*hs1, ei2d)
    return tail(*acc1, dinv, bg1r, Wfc1, bfc1r, g1r, be1r)
